# Initial kernel scaffold; baseline (speedup 1.0000x reference)
import jax
import jax.numpy as jnp
from jax.experimental import pallas as pl


def kernel(yh, y):
    raise NotImplementedError("write your pallas kernel here")


# speedup vs baseline: 3.0688x; 3.0688x over previous
import functools

import jax
import jax.numpy as jnp
from jax.experimental import pallas as pl

B = 16384
P = 294
BB = 2048


def _loss_kernel(yh_ref, y_ref, out_ref):
    i = pl.program_id(0)

    @pl.when(i == 0)
    def _init():
        out_ref[...] = jnp.zeros((1, 1), jnp.float32)

    v = yh_ref[...]
    yb = y_ref[...]
    j0 = (yb[:, 0:1] * 7.0 + yb[:, 1:2]).astype(jnp.int32)
    j1 = (yb[:, 4:5] * 7.0 + yb[:, 5:6]).astype(jnp.int32)
    t00 = yb[:, 2:3]
    t01 = yb[:, 3:4]
    t10 = yb[:, 6:7]
    t11 = yb[:, 7:8]

    pos = jax.lax.broadcasted_iota(jnp.int32, (BB, P), 1)
    dense = jnp.where((pos < 49) | ((pos >= 147) & (pos < 196)), 0.5, 0.0)
    onehot = (pos == j0) | (pos == (j1 + 147))

    term = dense * v * v
    term = term + jnp.where(onehot, (1.0 - v) * (1.0 - v) - 0.5 * v * v, 0.0)
    d1 = v - t00
    d2 = v - t01
    d4 = v - t10
    d5 = v - t11
    term = term + jnp.where(pos == (j0 + 49), 5.0 * d1 * d1, 0.0)
    term = term + jnp.where(pos == (j0 + 98), 5.0 * d2 * d2, 0.0)
    term = term + jnp.where(pos == (j1 + 196), 5.0 * d4 * d4, 0.0)
    term = term + jnp.where(pos == (j1 + 245), 5.0 * d5 * d5, 0.0)

    out_ref[...] += jnp.sum(term).reshape(1, 1)


@jax.jit
def kernel(yh, y):
    yh2 = yh.reshape(B, P)
    y2 = y.reshape(B, 8)
    out = pl.pallas_call(
        _loss_kernel,
        grid=(B // BB,),
        in_specs=[
            pl.BlockSpec((BB, P), lambda i: (i, 0)),
            pl.BlockSpec((BB, 8), lambda i: (i, 0)),
        ],
        out_specs=pl.BlockSpec((1, 1), lambda i: (0, 0)),
        out_shape=jax.ShapeDtypeStruct((1, 1), jnp.float32),
    )(yh2, y2)
    return out[0, 0]
